# Initial kernel scaffold; baseline (speedup 1.0000x reference)
import jax
import jax.numpy as jnp
from jax.experimental import pallas as pl


def kernel(x, edge_index, edge_attr, W1, b1, W2, b2, W3, b3, W4, b4, W5, b5, ln1_g, ln1_b, ln2_g, ln2_b):
    raise NotImplementedError("write your pallas kernel here")


# speedup vs baseline: 3.0523x; 3.0523x over previous
import functools

import jax
import jax.numpy as jnp
from jax import lax
from jax.experimental import pallas as pl
from jax.experimental.pallas import tpu as pltpu
from jax.experimental.pallas import tpu_sc as plsc

N = 10000
NP = 10240
E = 320000
D = 128
DH = D // 2
DE = 16
H = 4 * D

NC = 2
NS = 16
L = 16
EPW = E // NS
B = 80
CH = EPW // B
RPT = NP // NS
ZR = 128

_f32 = jnp.float32


def _fold_body(w1_ref, w2_ref, b1_ref, b2_ref, w12_ref, bb_ref):
    w2 = w2_ref[...]
    w12_ref[...] = jnp.dot(w1_ref[...], w2, preferred_element_type=_f32)
    bb_ref[...] = jnp.dot(b1_ref[...], w2, preferred_element_type=_f32) + b2_ref[...]


def _fold(W1, W2, b1, b2):
    return pl.pallas_call(
        _fold_body,
        out_shape=(
            jax.ShapeDtypeStruct((2 * D + DE, D), _f32),
            jax.ShapeDtypeStruct((1, D), _f32),
        ),
    )(W1, W2, b1.reshape(1, D), b2.reshape(1, D))


_PQ_BN = 1024


def _pq_body(x_ref, w12_ref, pa_ref, pb_ref, qa_ref, qb_ref):
    xb = x_ref[...]
    p = jnp.dot(xb, w12_ref[:D, :], preferred_element_type=_f32)
    q = jnp.dot(xb, w12_ref[D:2 * D, :], preferred_element_type=_f32)
    pa_ref[...] = p[:, :DH]
    pb_ref[...] = p[:, DH:]
    qa_ref[...] = q[:, :DH]
    qb_ref[...] = q[:, DH:]


def _pq(x, W12):
    half = jax.ShapeDtypeStruct((NP, DH), _f32)
    return pl.pallas_call(
        _pq_body,
        grid=(NP // _PQ_BN,),
        in_specs=[
            pl.BlockSpec((_PQ_BN, D), lambda i: (i, 0)),
            pl.BlockSpec((2 * D + DE, D), lambda i: (0, 0)),
        ],
        out_specs=tuple(
            pl.BlockSpec((_PQ_BN, DH), lambda i: (i, 0)) for _ in range(4)),
        out_shape=(half, half, half, half),
    )(x, W12)


_R_BN = 4000


def _r_body(ea_ref, w12_ref, bb_ref, ra_ref, rb_ref):
    r = (jnp.dot(ea_ref[...], w12_ref[2 * D:, :], preferred_element_type=_f32)
         + bb_ref[...])
    ra_ref[...] = r[:, :DH]
    rb_ref[...] = r[:, DH:]


def _r(edge_attr, W12, bb):
    half = jax.ShapeDtypeStruct((E, DH), _f32)
    return pl.pallas_call(
        _r_body,
        grid=(E // _R_BN,),
        in_specs=[
            pl.BlockSpec((_R_BN, DE), lambda i: (i, 0)),
            pl.BlockSpec((2 * D + DE, D), lambda i: (0, 0)),
            pl.BlockSpec((1, D), lambda i: (0, 0)),
        ],
        out_specs=(
            pl.BlockSpec((_R_BN, DH), lambda i: (i, 0)),
            pl.BlockSpec((_R_BN, DH), lambda i: (i, 0)),
        ),
        out_shape=(half, half),
    )(edge_attr, W12, bb)


def _sc_edge_body(pa_hbm, pb_hbm, qa_hbm, qb_hbm, ra_hbm, rb_hbm,
                  dst_hbm, src_hbm,
                  s0_out, s1_out, c0_out, c1_out,
                  dst_v, src_v, p_v, q_v, r_v, o_v, ones_v, z_v, zc_v,
                  s_sh, c_sh, sem_p, sem_q):
    c = lax.axis_index("c")
    s = lax.axis_index("s")
    base_row = s * RPT

    zeros16 = jnp.zeros((L,), _f32)
    ones16 = jnp.ones((L,), _f32)

    def _zrow(i, carry):
        for j in range(DH // L):
            z_v[i, pl.ds(j * L, L)] = zeros16
        zc_v[i, :] = zeros16
        return carry

    lax.fori_loop(0, ZR, _zrow, 0)

    def _orow(i, carry):
        ones_v[i, :] = ones16
        return carry

    lax.fori_loop(0, B, _orow, 0)

    for k in range(RPT // ZR):
        pltpu.sync_copy(z_v, s_sh.at[pl.ds(base_row + k * ZR, ZR)])
        pltpu.sync_copy(zc_v, c_sh.at[pl.ds(base_row + k * ZR, ZR)])

    pltpu.sync_copy(dst_hbm.at[s], dst_v)
    pltpu.sync_copy(src_hbm.at[s], src_v)

    plsc.subcore_barrier()

    def _make_chunk(p_hbm, q_hbm, r_hbm, count_parity):
        def _chunk(ci, carry):
            eb = s * EPW + ci * B
            cp = pltpu.async_copy(p_hbm.at[dst_v.at[ci]], p_v, sem_p)
            cq = pltpu.async_copy(q_hbm.at[src_v.at[ci]], q_v, sem_q)
            pltpu.sync_copy(r_hbm.at[pl.ds(eb, B)], r_v)
            cp.wait()
            cq.wait()

            def _edge(i, icarry):
                for j in range(DH // L):
                    sl = pl.ds(j * L, L)
                    m = p_v[i, sl] + q_v[i, sl] + r_v[i, sl]
                    o_v[i, sl] = 1.0 / (1.0 + jnp.exp(-m))
                return icarry

            lax.fori_loop(0, B, _edge, 0)

            pltpu.sync_copy(o_v, s_sh.at[dst_v.at[ci]], add=True)

            @pl.when(lax.rem(ci, 2) == count_parity)
            def _():
                pltpu.sync_copy(ones_v, c_sh.at[dst_v.at[ci]], add=True)

            return carry
        return _chunk

    @pl.when(c == 0)
    def _():
        lax.fori_loop(0, CH, _make_chunk(pa_hbm, qa_hbm, ra_hbm, 0), 0)

    @pl.when(c == 1)
    def _():
        lax.fori_loop(0, CH, _make_chunk(pb_hbm, qb_hbm, rb_hbm, 1), 0)

    plsc.subcore_barrier()

    sl = pl.ds(base_row, RPT)

    @pl.when(c == 0)
    def _():
        pltpu.sync_copy(s_sh.at[sl], s0_out.at[sl])
        pltpu.sync_copy(c_sh.at[sl], c0_out.at[sl])

    @pl.when(c == 1)
    def _():
        pltpu.sync_copy(s_sh.at[sl], s1_out.at[sl])
        pltpu.sync_copy(c_sh.at[sl], c1_out.at[sl])


def _sc_edge(Pa, Pb, Qa, Qb, Ra, Rb, dst3d, src3d):
    mesh = plsc.VectorSubcoreMesh(core_axis_name="c", subcore_axis_name="s")
    fn = functools.partial(
        pl.kernel,
        out_type=(
            jax.ShapeDtypeStruct((NP, DH), _f32),
            jax.ShapeDtypeStruct((NP, DH), _f32),
            jax.ShapeDtypeStruct((NP, L), _f32),
            jax.ShapeDtypeStruct((NP, L), _f32),
        ),
        mesh=mesh,
        compiler_params=pltpu.CompilerParams(use_tc_tiling_on_sc=False),
        scratch_types=[
            pltpu.VMEM((CH, B), jnp.int32),
            pltpu.VMEM((CH, B), jnp.int32),
            pltpu.VMEM((B, DH), _f32),
            pltpu.VMEM((B, DH), _f32),
            pltpu.VMEM((B, DH), _f32),
            pltpu.VMEM((B, DH), _f32),
            pltpu.VMEM((B, L), _f32),
            pltpu.VMEM((ZR, DH), _f32),
            pltpu.VMEM((ZR, L), _f32),
            pltpu.VMEM_SHARED((NP, DH), _f32),
            pltpu.VMEM_SHARED((NP, L), _f32),
            pltpu.SemaphoreType.DMA,
            pltpu.SemaphoreType.DMA,
        ],
    )(_sc_edge_body)
    return fn(Pa, Pb, Qa, Qb, Ra, Rb, dst3d, src3d)


_PO_BN = 1000


def _ln(h, g, b):
    mu = jnp.mean(h, axis=-1, keepdims=True)
    zc = h - mu
    var = jnp.mean(zc * zc, axis=-1, keepdims=True)
    return zc * lax.rsqrt(var + 1e-5) * g + b


def _post_body(x_ref, s0_ref, s1_ref, c0_ref, c1_ref,
               w3_ref, b3_ref, w4_ref, b4_ref, w5_ref, b5_ref,
               g1_ref, bb1_ref, g2_ref, bb2_ref, out_ref):
    S = jnp.concatenate([s0_ref[...], s1_ref[...]], axis=-1)
    cnt = c0_ref[...][:, :1] + c1_ref[...][:, :1]
    A = jnp.dot(S, w3_ref[...], preferred_element_type=_f32)
    dh = (A + cnt * b3_ref[...]) / jnp.maximum(cnt, 1.0)
    h = _ln(x_ref[...] + dh, g1_ref[...], bb1_ref[...])
    t = jnp.dot(h, w4_ref[...], preferred_element_type=_f32) + b4_ref[...]
    hid = 1.0 / (1.0 + jnp.exp(-t))
    ff = jnp.dot(hid, w5_ref[...], preferred_element_type=_f32) + b5_ref[...]
    out_ref[...] = _ln(h + ff, g2_ref[...], bb2_ref[...])


def _post(x, S0, S1, C0, C1, W3, b3, W4, b4, W5, b5, g1, bb1, g2, bb2):
    row = lambda i: (i, 0)
    const2 = lambda shape: pl.BlockSpec(shape, lambda i: (0, 0))
    return pl.pallas_call(
        _post_body,
        grid=(N // _PO_BN,),
        in_specs=[
            pl.BlockSpec((_PO_BN, D), row),
            pl.BlockSpec((_PO_BN, DH), row),
            pl.BlockSpec((_PO_BN, DH), row),
            pl.BlockSpec((_PO_BN, L), row),
            pl.BlockSpec((_PO_BN, L), row),
            const2((D, D)),
            const2((1, D)),
            const2((D, H)),
            const2((1, H)),
            const2((H, D)),
            const2((1, D)),
            const2((1, D)),
            const2((1, D)),
            const2((1, D)),
            const2((1, D)),
        ],
        out_specs=pl.BlockSpec((_PO_BN, D), row),
        out_shape=jax.ShapeDtypeStruct((N, D), _f32),
    )(x, S0, S1, C0, C1, W3, b3.reshape(1, D), W4, b4.reshape(1, H),
      W5, b5.reshape(1, D), g1.reshape(1, D), bb1.reshape(1, D),
      g2.reshape(1, D), bb2.reshape(1, D))


def kernel(x, edge_index, edge_attr, W1, b1, W2, b2, W3, b3, W4, b4, W5, b5,
           ln1_g, ln1_b, ln2_g, ln2_b):
    W12, bb = _fold(W1, W2, b1, b2)
    xp = jnp.pad(x, ((0, NP - N), (0, 0)))
    Pa, Pb, Qa, Qb = _pq(xp, W12)
    Ra, Rb = _r(edge_attr, W12, bb)
    src = edge_index[0].astype(jnp.int32).reshape(NS, CH, B)
    dst = edge_index[1].astype(jnp.int32).reshape(NS, CH, B)
    S0, S1, C0, C1 = _sc_edge(Pa, Pb, Qa, Qb, Ra, Rb, dst, src)
    return _post(x, S0, S1, C0, C1, W3, b3, W4, b4, W5, b5,
                 ln1_g, ln1_b, ln2_g, ln2_b)
